# baseline (speedup 1.0000x reference)
import functools

import jax
import jax.numpy as jnp
import numpy as np
from jax import lax
from jax.experimental import pallas as pl
from jax.experimental.pallas import tpu as pltpu
from jax.experimental.pallas import tpu_sc as plsc

B = 16384
D = 32
NCLS = (3, 6, 9, 38)
NLEVELS = 4
T_ROWS = 3 * 6 * 9 * 38
T_PAD = 6272
NC, NS = 2, 16
NW = NC * NS
BPW = B // NW
LANES = 16


_KTOT = sum(NCLS)


_TBLK = T_PAD // 8


def _multihot_const() -> jnp.ndarray:
    o = np.zeros((T_PAD, _KTOT), np.float32)
    j = np.arange(T_ROWS)
    offs = np.cumsum([0] + list(NCLS))[:4]
    divs = (1, 3, 18, 162)
    for l in range(NLEVELS):
        o[j, offs[l] + (j // divs[l]) % NCLS[l]] = 1.0
    return jnp.asarray(o, dtype=jnp.bfloat16)


def _tc_build_body(o_ref, wbd_ref, t_ref):
    t_ref[...] = jax.lax.dot_general(
        o_ref[...], wbd_ref[...].astype(jnp.bfloat16), (((1,), (0,)), ((), ())),
        preferred_element_type=jnp.float32)


_tc_build = pl.pallas_call(
    _tc_build_body,
    grid=(8,),
    in_specs=[
        pl.BlockSpec((_TBLK, _KTOT), lambda i: (i, 0)),
        pl.BlockSpec((_KTOT, NLEVELS * D), lambda i: (0, 0)),
    ],
    out_specs=pl.BlockSpec((_TBLK, NLEVELS * D), lambda i: (i, 0)),
    out_shape=jax.ShapeDtypeStruct((T_PAD, NLEVELS * D), jnp.float32),
)


def _make_sc_kernel():
    mesh = plsc.VectorSubcoreMesh(core_axis_name="c", subcore_axis_name="s")

    @pl.kernel(
        out_type=jax.ShapeDtypeStruct((B, NLEVELS * D), jnp.float32),
        mesh=mesh,
        scratch_types=(
            [pltpu.VMEM((BPW,), jnp.int32) for _ in range(NLEVELS + 1)]
            + [pltpu.VMEM((BPW, NLEVELS * D), jnp.float32)]
            + [pltpu.SemaphoreType.DMA, pltpu.SemaphoreType.DMA]
        ),
    )
    def sc_kernel(i0_hbm, i1_hbm, i2_hbm, i3_hbm, t_hbm, out_hbm,
                  iv0, iv1, iv2, iv3, ivc, rows, sem_i, sem_g):
        wid = lax.axis_index("s") * NC + lax.axis_index("c")
        base = wid * BPW
        idx_hbms = [i0_hbm, i1_hbm, i2_hbm, i3_hbm]
        ivs = [iv0, iv1, iv2, iv3]

        idx_copies = [
            pltpu.async_copy(idx_hbms[l].at[pl.ds(base, BPW)], ivs[l], sem_i)
            for l in range(NLEVELS)
        ]
        for c in idx_copies:
            c.wait()

        @pl.loop(0, BPW, step=LANES)
        def _(i):
            s = pl.ds(i, LANES)
            ivc[s] = ((iv3[s] * 9 + iv2[s]) * 6 + iv1[s]) * 3 + iv0[s]

        pltpu.async_copy(t_hbm.at[ivc], rows, sem_g).wait()
        pltpu.sync_copy(rows, out_hbm.at[pl.ds(base, BPW)])

    return sc_kernel


_sc_kernel = _make_sc_kernel()


def kernel(idx0, idx1, idx2, idx3, W0, W1, W2, W3):
    wbd = jnp.concatenate([
        jnp.pad(W0, ((0, 0), (0, 96))),
        jnp.pad(W1, ((0, 0), (32, 64))),
        jnp.pad(W2, ((0, 0), (64, 32))),
        jnp.pad(W3, ((0, 0), (96, 0))),
    ], axis=0)
    t = _tc_build(_multihot_const(), wbd)
    flat = _sc_kernel(idx0.astype(jnp.int32), idx1.astype(jnp.int32),
                      idx2.astype(jnp.int32), idx3.astype(jnp.int32), t)
    return flat.reshape(B, 1, NLEVELS * D)


# speedup vs baseline: 1.1356x; 1.1356x over previous
import functools

import jax
import jax.numpy as jnp
import numpy as np
from jax import lax
from jax.experimental import pallas as pl
from jax.experimental.pallas import tpu as pltpu
from jax.experimental.pallas import tpu_sc as plsc

B = 16384
D = 32
NCLS = (3, 6, 9, 38)
NLEVELS = 4
T_ROWS = 3 * 6 * 9 * 38
T_PAD = 6272
NC, NS = 2, 16
NW = NC * NS
BPW = B // NW
LANES = 16


_KTOT = sum(NCLS)


_TBLK = T_PAD // 8


def _multihot_const() -> jnp.ndarray:
    o = np.zeros((T_PAD, _KTOT), np.float32)
    j = np.arange(T_ROWS)
    offs = np.cumsum([0] + list(NCLS))[:4]
    divs = (1, 3, 18, 162)
    for l in range(NLEVELS):
        o[j, offs[l] + (j // divs[l]) % NCLS[l]] = 1.0
    return jnp.asarray(o, dtype=jnp.bfloat16)


def _tc_build_body(o_ref, w0_ref, w1_ref, w2_ref, w3_ref, t_ref, wbd_ref):
    wbd_ref[...] = jnp.zeros((_KTOT, NLEVELS * D), jnp.bfloat16)
    wbd_ref[0:3, 0:32] = w0_ref[...].astype(jnp.bfloat16)
    wbd_ref[3:9, 32:64] = w1_ref[...].astype(jnp.bfloat16)
    wbd_ref[9:18, 64:96] = w2_ref[...].astype(jnp.bfloat16)
    wbd_ref[18:56, 96:128] = w3_ref[...].astype(jnp.bfloat16)
    t_ref[...] = jax.lax.dot_general(
        o_ref[...], wbd_ref[...], (((1,), (0,)), ((), ())),
        preferred_element_type=jnp.float32)


_tc_build = pl.pallas_call(
    _tc_build_body,
    out_shape=jax.ShapeDtypeStruct((T_PAD, NLEVELS * D), jnp.float32),
    scratch_shapes=[pltpu.VMEM((_KTOT, NLEVELS * D), jnp.bfloat16)],
)


def _make_sc_kernel():
    mesh = plsc.VectorSubcoreMesh(core_axis_name="c", subcore_axis_name="s")

    @pl.kernel(
        out_type=jax.ShapeDtypeStruct((B, NLEVELS * D), jnp.float32),
        mesh=mesh,
        scratch_types=(
            [pltpu.VMEM((BPW,), jnp.int32) for _ in range(NLEVELS + 1)]
            + [pltpu.VMEM((BPW, NLEVELS * D), jnp.float32)]
            + [pltpu.SemaphoreType.DMA, pltpu.SemaphoreType.DMA]
        ),
    )
    def sc_kernel(i0_hbm, i1_hbm, i2_hbm, i3_hbm, t_hbm, out_hbm,
                  iv0, iv1, iv2, iv3, ivc, rows, sem_i, sem_g):
        wid = lax.axis_index("s") * NC + lax.axis_index("c")
        base = wid * BPW
        idx_hbms = [i0_hbm, i1_hbm, i2_hbm, i3_hbm]
        ivs = [iv0, iv1, iv2, iv3]

        idx_copies = [
            pltpu.async_copy(idx_hbms[l].at[pl.ds(base, BPW)], ivs[l], sem_i)
            for l in range(NLEVELS)
        ]
        for c in idx_copies:
            c.wait()

        @pl.loop(0, BPW, step=LANES)
        def _(i):
            s = pl.ds(i, LANES)
            ivc[s] = ((iv3[s] * 9 + iv2[s]) * 6 + iv1[s]) * 3 + iv0[s]

        pltpu.async_copy(t_hbm.at[ivc], rows, sem_g).wait()
        pltpu.sync_copy(rows, out_hbm.at[pl.ds(base, BPW)])

    return sc_kernel


_sc_kernel = _make_sc_kernel()


def kernel(idx0, idx1, idx2, idx3, W0, W1, W2, W3):
    t = _tc_build(_multihot_const(), W0, W1, W2, W3)
    flat = _sc_kernel(idx0.astype(jnp.int32), idx1.astype(jnp.int32),
                      idx2.astype(jnp.int32), idx3.astype(jnp.int32), t)
    return flat.reshape(B, 1, NLEVELS * D)
